# baseline (speedup 1.0000x reference)
import functools

import jax
import jax.numpy as jnp
from jax import lax
from jax.experimental import pallas as pl
from jax.experimental.pallas import tpu as pltpu
from jax.experimental.pallas import tpu_sc as plsc

NUM_CORES = 2
NUM_SUBCORES = 16
NUM_WORKERS = NUM_CORES * NUM_SUBCORES


@functools.partial(jax.jit, static_argnums=(2, 3))
def _sc_gather(idx_flat, table, chunk, d):
    n_rows = idx_flat.shape[0]
    rows_per_worker = n_rows // NUM_WORKERS
    n_chunks = rows_per_worker // chunk
    assert rows_per_worker % chunk == 0

    mesh = plsc.VectorSubcoreMesh(
        core_axis_name="c",
        subcore_axis_name="s",
        num_cores=NUM_CORES,
        num_subcores=NUM_SUBCORES,
    )

    @functools.partial(
        pl.kernel,
        out_type=jax.ShapeDtypeStruct((n_rows, d), jnp.float32),
        mesh=mesh,
        scratch_types=[
            pltpu.VMEM((chunk,), jnp.int32),
            pltpu.VMEM((chunk, d), jnp.float32),
            pltpu.SemaphoreType.DMA,
            pltpu.SemaphoreType.DMA,
            pltpu.SemaphoreType.DMA,
        ],
        compiler_params=pltpu.CompilerParams(use_tc_tiling_on_sc=False),
    )
    def k(idx_hbm, table_hbm, out_hbm, idx_v, rows_v, idx_sem, gat_sem, out_sem):
        wid = lax.axis_index("s") * NUM_CORES + lax.axis_index("c")
        base = wid * rows_per_worker

        def body(g):
            start = base + g * chunk
            pltpu.async_copy(idx_hbm.at[pl.ds(start, chunk)], idx_v, idx_sem).wait()
            pltpu.async_copy(table_hbm.at[idx_v], rows_v, gat_sem).wait()
            pltpu.async_copy(rows_v, out_hbm.at[pl.ds(start, chunk)], out_sem).wait()

        pl.loop(0, n_chunks)(body)

    return k(idx_flat, table)


def kernel(idxs, entity_embeds):
    batch, hist = idxs.shape
    d = entity_embeds.shape[1]
    flat = idxs.reshape(-1)
    out = _sc_gather(flat, entity_embeds, 1024, d)
    return out.reshape(batch, hist, d)


# speedup vs baseline: 1.0311x; 1.0311x over previous
import functools

import jax
import jax.numpy as jnp
from jax import lax
from jax.experimental import pallas as pl
from jax.experimental.pallas import tpu as pltpu
from jax.experimental.pallas import tpu_sc as plsc

NUM_CORES = 2
NUM_SUBCORES = 16
NUM_WORKERS = NUM_CORES * NUM_SUBCORES


@functools.partial(jax.jit, static_argnums=(2, 3))
def _sc_gather(idx_flat, table, chunk, d):
    n_rows = idx_flat.shape[0]
    rows_per_worker = n_rows // NUM_WORKERS
    n_chunks = rows_per_worker // chunk
    assert rows_per_worker % chunk == 0 and n_chunks % 2 == 0 and n_chunks >= 4

    mesh = plsc.VectorSubcoreMesh(
        core_axis_name="c",
        subcore_axis_name="s",
        num_cores=NUM_CORES,
        num_subcores=NUM_SUBCORES,
    )

    @functools.partial(
        pl.kernel,
        out_type=jax.ShapeDtypeStruct((n_rows, d), jnp.float32),
        mesh=mesh,
        scratch_types=[
            pltpu.VMEM((chunk,), jnp.int32),
            pltpu.VMEM((chunk,), jnp.int32),
            pltpu.VMEM((chunk, d), jnp.float32),
            pltpu.VMEM((chunk, d), jnp.float32),
            pltpu.SemaphoreType.DMA,
            pltpu.SemaphoreType.DMA,
            pltpu.SemaphoreType.DMA,
            pltpu.SemaphoreType.DMA,
            pltpu.SemaphoreType.DMA,
            pltpu.SemaphoreType.DMA,
        ],
        compiler_params=pltpu.CompilerParams(use_tc_tiling_on_sc=False),
    )
    def k(idx_hbm, table_hbm, out_hbm, idx_v0, idx_v1, rows_v0, rows_v1,
          idx_sem0, idx_sem1, gat_sem0, gat_sem1, out_sem0, out_sem1):
        wid = lax.axis_index("s") * NUM_CORES + lax.axis_index("c")
        base = wid * rows_per_worker

        idx_v = (idx_v0, idx_v1)
        rows_v = (rows_v0, rows_v1)
        idx_sem = (idx_sem0, idx_sem1)
        gat_sem = (gat_sem0, gat_sem1)
        out_sem = (out_sem0, out_sem1)

        def start_idx(g, s):
            pltpu.async_copy(
                idx_hbm.at[pl.ds(base + g * chunk, chunk)], idx_v[s], idx_sem[s])

        def wait_idx(s):
            pltpu.make_async_copy(
                idx_hbm.at[pl.ds(base, chunk)], idx_v[s], idx_sem[s]).wait()

        def start_gather(s):
            pltpu.async_copy(table_hbm.at[idx_v[s]], rows_v[s], gat_sem[s])

        def wait_gather(s):
            pltpu.make_async_copy(
                table_hbm.at[idx_v[s]], rows_v[s], gat_sem[s]).wait()

        def start_store(g, s):
            pltpu.async_copy(
                rows_v[s], out_hbm.at[pl.ds(base + g * chunk, chunk)], out_sem[s])

        def wait_store(s):
            pltpu.make_async_copy(
                rows_v[s], out_hbm.at[pl.ds(base, chunk)], out_sem[s]).wait()

        start_idx(0, 0)
        start_idx(1, 1)

        def chunk_body(g, s):
            pl.when(g >= 2)(lambda: wait_store(s))
            wait_idx(s)
            start_gather(s)
            def drain_prev():
                wait_gather(1 - s)
                start_store(g - 1, 1 - s)
                pl.when(g + 1 < n_chunks)(lambda: start_idx(g + 1, 1 - s))
            pl.when(g >= 1)(drain_prev)

        def outer(i):
            chunk_body(i * 2, 0)
            chunk_body(i * 2 + 1, 1)

        pl.loop(0, n_chunks // 2)(outer)

        last = (n_chunks - 1) % 2
        wait_gather(last)
        start_store(n_chunks - 1, last)
        wait_store(1 - last)
        wait_store(last)

    return k(idx_flat, table)


def kernel(idxs, entity_embeds):
    batch, hist = idxs.shape
    d = entity_embeds.shape[1]
    flat = idxs.reshape(-1)
    out = _sc_gather(flat, entity_embeds, 800, d)
    return out.reshape(batch, hist, d)
